# baseline (speedup 1.0000x reference)
import functools

import jax
import jax.numpy as jnp
from jax import lax
from jax.experimental import pallas as pl
from jax.experimental.pallas import tpu as pltpu
from jax.experimental.pallas import tpu_sc as plsc

N_SPANS = 16384
DIMS = 384
SPAN_DIM = 128
N_DOCS = 150
N_POS = 4098
G_ROWS = 4104
DOC_ROW = N_POS

NW = 32
SPANS_PER_W = N_SPANS // NW
CHUNK = 128
NCHUNK = SPANS_PER_W // CHUNK


def _prep_body(doc_id_ref, doc_ref, span_ref, g_ref):
    g_ref[pl.ds(0, N_POS), :] = jnp.tanh(span_ref[...]) * 0.1
    row = doc_ref[pl.ds(doc_id_ref[0], 1), :]
    g_ref[pl.ds(DOC_ROW, 1), :] = jnp.tanh(row) * 0.1
    g_ref[pl.ds(DOC_ROW + 1, G_ROWS - DOC_ROW - 1), :] = jnp.zeros(
        (G_ROWS - DOC_ROW - 1, SPAN_DIM), jnp.float32)


def _prep_table(doc_id_vec, doc_table, span_table):
    return pl.pallas_call(
        _prep_body,
        out_shape=jax.ShapeDtypeStruct((G_ROWS, SPAN_DIM), jnp.float32),
        in_specs=[
            pl.BlockSpec(memory_space=pltpu.SMEM),
            pl.BlockSpec(memory_space=pltpu.VMEM),
            pl.BlockSpec(memory_space=pltpu.VMEM),
        ],
        out_specs=pl.BlockSpec(memory_space=pltpu.VMEM),
    )(doc_id_vec, doc_table, span_table)


def _sc_body(g_hbm, starts_hbm, ends_hbm, out_hbm,
             sv, ev, di, bufd, bufs0, bufs1, bufe0, bufe1,
             semd, sems0, sems1, seme0, seme1):
    wid = lax.axis_index("s") * 2 + lax.axis_index("c")
    base = wid * SPANS_PER_W
    pltpu.sync_copy(starts_hbm.at[pl.ds(base, SPANS_PER_W)], sv)
    pltpu.sync_copy(ends_hbm.at[pl.ds(base, SPANS_PER_W)], ev)

    doc_fill = jnp.full((16,), DOC_ROW, jnp.int32)
    for j in range(CHUNK // 16):
        di[pl.ds(j * 16, 16)] = doc_fill
    hd = pltpu.async_copy(g_hbm.at[di], bufd, semd)

    sbufs = (bufs0, bufs1)
    ebufs = (bufe0, bufe1)
    ssems = (sems0, sems1)
    esems = (seme0, seme1)

    def gather_start(t, b):
        sl = pl.ds(t * CHUNK, CHUNK)
        return (pltpu.async_copy(g_hbm.at[sv.at[sl]], sbufs[b], ssems[b]),
                pltpu.async_copy(g_hbm.at[ev.at[sl]], ebufs[b], esems[b]))

    handles = [None] * NCHUNK
    handles[0] = gather_start(0, 0)
    handles[1] = gather_start(1, 1)
    hd.wait()
    for t in range(NCHUNK):
        row0 = 3 * base + t * 3 * CHUNK
        hs, he = handles[t]
        pltpu.sync_copy(bufd, out_hbm.at[pl.ds(row0, CHUNK)])
        hs.wait()
        pltpu.sync_copy(sbufs[t % 2], out_hbm.at[pl.ds(row0 + CHUNK, CHUNK)])
        he.wait()
        pltpu.sync_copy(ebufs[t % 2], out_hbm.at[pl.ds(row0 + 2 * CHUNK, CHUNK)])
        if t + 2 < NCHUNK:
            handles[t + 2] = gather_start(t + 2, t % 2)


def _sc_gather(g, starts, ends):
    mesh = plsc.VectorSubcoreMesh(core_axis_name="c", subcore_axis_name="s")
    kern = functools.partial(
        pl.kernel,
        mesh=mesh,
        out_type=jax.ShapeDtypeStruct((3 * N_SPANS, SPAN_DIM), jnp.float32),
        scratch_types=[
            pltpu.VMEM((SPANS_PER_W,), jnp.int32),
            pltpu.VMEM((SPANS_PER_W,), jnp.int32),
            pltpu.VMEM((CHUNK,), jnp.int32),
            pltpu.VMEM((CHUNK, SPAN_DIM), jnp.float32),
            pltpu.VMEM((CHUNK, SPAN_DIM), jnp.float32),
            pltpu.VMEM((CHUNK, SPAN_DIM), jnp.float32),
            pltpu.VMEM((CHUNK, SPAN_DIM), jnp.float32),
            pltpu.VMEM((CHUNK, SPAN_DIM), jnp.float32),
            pltpu.SemaphoreType.DMA,
            pltpu.SemaphoreType.DMA,
            pltpu.SemaphoreType.DMA,
            pltpu.SemaphoreType.DMA,
            pltpu.SemaphoreType.DMA,
        ],
    )(_sc_body)
    return kern(g, starts, ends)


def kernel(doc_id, span_starts, span_ends, doc_table, span_table):
    doc_id_vec = jnp.reshape(jnp.asarray(doc_id, jnp.int32), (1,))
    g = _prep_table(doc_id_vec, doc_table, span_table)
    rows = _sc_gather(g,
                      span_starts.astype(jnp.int32),
                      span_ends.astype(jnp.int32))
    return rows.reshape(N_SPANS, DIMS)


# speedup vs baseline: 6.5779x; 6.5779x over previous
import functools

import jax
import jax.numpy as jnp
from jax import lax
from jax.experimental import pallas as pl
from jax.experimental.pallas import tpu as pltpu
from jax.experimental.pallas import tpu_sc as plsc

N_SPANS = 16384
DIMS = 384
SPAN_DIM = 128
N_DOCS = 150
N_POS = 4098
G_ROWS = 4224
DOC_ROW = N_POS

NW = 32
NS = 16
STAGE_ROWS = G_ROWS // NS
SPANS_PER_W = N_SPANS // NW
CHUNK = 128
NCHUNK = SPANS_PER_W // CHUNK


def _prep_body(doc_id_ref, doc_ref, span_ref, g_ref):
    g_ref[pl.ds(0, N_POS), :] = jnp.tanh(span_ref[...]) * 0.1
    row = doc_ref[pl.ds(doc_id_ref[0], 1), :]
    g_ref[pl.ds(DOC_ROW, 1), :] = jnp.tanh(row) * 0.1
    g_ref[pl.ds(DOC_ROW + 1, G_ROWS - DOC_ROW - 1), :] = jnp.zeros(
        (G_ROWS - DOC_ROW - 1, SPAN_DIM), jnp.float32)


def _prep_table(doc_id_vec, doc_table, span_table):
    return pl.pallas_call(
        _prep_body,
        out_shape=jax.ShapeDtypeStruct((G_ROWS, SPAN_DIM), jnp.float32),
        in_specs=[
            pl.BlockSpec(memory_space=pltpu.SMEM),
            pl.BlockSpec(memory_space=pltpu.VMEM),
            pl.BlockSpec(memory_space=pltpu.VMEM),
        ],
        out_specs=pl.BlockSpec(memory_space=pltpu.VMEM),
    )(doc_id_vec, doc_table, span_table)


def _sc_body(g_hbm, starts_hbm, ends_hbm, out_hbm,
             gsh, sv, ev, di, bufd, bufs0, bufs1, bufe0, bufe1,
             semd, sems0, sems1, seme0, seme1):
    sid = lax.axis_index("s")
    wid = sid * 2 + lax.axis_index("c")
    base = wid * SPANS_PER_W

    stage = pl.ds(sid * STAGE_ROWS, STAGE_ROWS)
    pltpu.sync_copy(g_hbm.at[stage], gsh.at[stage])

    pltpu.sync_copy(starts_hbm.at[pl.ds(base, SPANS_PER_W)], sv)
    pltpu.sync_copy(ends_hbm.at[pl.ds(base, SPANS_PER_W)], ev)

    doc_fill = jnp.full((16,), DOC_ROW, jnp.int32)
    for j in range(CHUNK // 16):
        di[pl.ds(j * 16, 16)] = doc_fill

    plsc.subcore_barrier()
    hd = pltpu.async_copy(gsh.at[di], bufd, semd)

    sbufs = (bufs0, bufs1)
    ebufs = (bufe0, bufe1)
    ssems = (sems0, sems1)
    esems = (seme0, seme1)

    def gather_start(t, b):
        sl = pl.ds(t * CHUNK, CHUNK)
        return (pltpu.async_copy(gsh.at[sv.at[sl]], sbufs[b], ssems[b]),
                pltpu.async_copy(gsh.at[ev.at[sl]], ebufs[b], esems[b]))

    handles = [None] * NCHUNK
    handles[0] = gather_start(0, 0)
    handles[1] = gather_start(1, 1)
    hd.wait()
    for t in range(NCHUNK):
        rows = pl.ds(base + t * CHUNK, CHUNK)
        hs, he = handles[t]
        pltpu.sync_copy(bufd, out_hbm.at[rows, pl.ds(0, SPAN_DIM)])
        hs.wait()
        pltpu.sync_copy(sbufs[t % 2], out_hbm.at[rows, pl.ds(SPAN_DIM, SPAN_DIM)])
        he.wait()
        pltpu.sync_copy(ebufs[t % 2], out_hbm.at[rows, pl.ds(2 * SPAN_DIM, SPAN_DIM)])
        if t + 2 < NCHUNK:
            handles[t + 2] = gather_start(t + 2, t % 2)


def _sc_gather(g, starts, ends):
    mesh = plsc.VectorSubcoreMesh(core_axis_name="c", subcore_axis_name="s")
    kern = functools.partial(
        pl.kernel,
        mesh=mesh,
        out_type=jax.ShapeDtypeStruct((N_SPANS, DIMS), jnp.float32),
        scratch_types=[
            pltpu.VMEM_SHARED((G_ROWS, SPAN_DIM), jnp.float32),
            pltpu.VMEM((SPANS_PER_W,), jnp.int32),
            pltpu.VMEM((SPANS_PER_W,), jnp.int32),
            pltpu.VMEM((CHUNK,), jnp.int32),
            pltpu.VMEM((CHUNK, SPAN_DIM), jnp.float32),
            pltpu.VMEM((CHUNK, SPAN_DIM), jnp.float32),
            pltpu.VMEM((CHUNK, SPAN_DIM), jnp.float32),
            pltpu.VMEM((CHUNK, SPAN_DIM), jnp.float32),
            pltpu.VMEM((CHUNK, SPAN_DIM), jnp.float32),
            pltpu.SemaphoreType.DMA,
            pltpu.SemaphoreType.DMA,
            pltpu.SemaphoreType.DMA,
            pltpu.SemaphoreType.DMA,
            pltpu.SemaphoreType.DMA,
        ],
    )(_sc_body)
    return kern(g, starts, ends)


def kernel(doc_id, span_starts, span_ends, doc_table, span_table):
    doc_id_vec = jnp.reshape(jnp.asarray(doc_id, jnp.int32), (1,))
    g = _prep_table(doc_id_vec, doc_table, span_table)
    return _sc_gather(g,
                      span_starts.astype(jnp.int32),
                      span_ends.astype(jnp.int32))
